# baseline (speedup 1.0000x reference)
import functools

import jax
import jax.numpy as jnp
from jax import lax
from jax.experimental import pallas as pl
from jax.experimental.pallas import tpu as pltpu
from jax.experimental.pallas import tpu_sc as plsc

N_NODES = 10000
N_EDGES = 320000
D = 128

NC = 2
NS = 16
DH = D // NC
EPT = N_EDGES // NS
CH = 40
K = 5
NCH = EPT // CH
NSTG = 10
NCH_S = NCH // NSTG
NBH_S = NCH_S // K
ACC_ROWS = N_NODES
ZPT = 632
LAST = N_NODES - (NS - 1) * ZPT
LANES = 16


def _make_sc_round(with_deg):
  mesh = plsc.VectorSubcoreMesh(core_axis_name="c", subcore_axis_name="s")

  sum_t = jax.ShapeDtypeStruct((NC, N_NODES, DH), jnp.float32)
  out_type = (sum_t, sum_t)
  scratch = [
      pltpu.VMEM((NCH_S, CH), jnp.int32),
      pltpu.VMEM((NCH_S, CH), jnp.int32),
      pltpu.VMEM((2 * K * CH, DH), jnp.float32),
      pltpu.VMEM_SHARED((ACC_ROWS, DH), jnp.float32),
      pltpu.VMEM_SHARED((ACC_ROWS, DH), jnp.float32),
      pltpu.SemaphoreType.DMA,
      pltpu.SemaphoreType.DMA,
  ]
  if with_deg:
    out_type = out_type + (
        jax.ShapeDtypeStruct((NC, N_NODES, 8), jnp.float32),)
    scratch = scratch + [
        pltpu.VMEM((CH, 8), jnp.float32),
        pltpu.VMEM_SHARED((ACC_ROWS, 8), jnp.float32),
        pltpu.SemaphoreType.DMA,
    ]

  @functools.partial(
      pl.kernel, out_type=out_type, mesh=mesh, scratch_types=scratch,
      compiler_params=pltpu.CompilerParams(use_tc_tiling_on_sc=False))
  def rnd(a_tab_h, p_tab_h, src_h, dst_h, *refs):
    if with_deg:
      (ones_h, zeros_h, sp_h, sa_h, deg_h, srcv, dstv, rows_v, acc_sh,
       tab_sh, gsem, ssem, ones_v, deg_sh, dsem) = refs
    else:
      sp_h, sa_h, srcv, dstv, rows_v, acc_sh, tab_sh, gsem, ssem = refs
    cid = lax.axis_index("c")
    sid = lax.axis_index("s")

    zvec = jnp.zeros((LANES,), jnp.float32)
    base = sid * ZPT

    def zero_acc():
      def zrow(r, carry):
        for j in range(DH // LANES):
          rows_v[r, pl.ds(j * LANES, LANES)] = zvec
        return carry

      lax.fori_loop(0, ZPT, zrow, 0)

      @pl.when(sid < NS - 1)
      def _zfull():
        pltpu.sync_copy(rows_v.at[pl.ds(0, ZPT)],
                        acc_sh.at[pl.ds(base, ZPT)])

      @pl.when(sid == NS - 1)
      def _zlast():
        pltpu.sync_copy(rows_v.at[pl.ds(0, LAST)],
                        acc_sh.at[pl.ds((NS - 1) * ZPT, LAST)])

    zero_acc()
    if with_deg:
      pltpu.sync_copy(zeros_h, ones_v)

      @pl.when(sid < NS - 1)
      def _dzfull():
        for rep in range(ZPT // CH):
          pltpu.sync_copy(ones_v, deg_sh.at[pl.ds(base + rep * CH, CH)])
        rem = ZPT - (ZPT // CH) * CH
        if rem:
          pltpu.sync_copy(ones_v.at[pl.ds(0, rem)],
                          deg_sh.at[pl.ds(base + (ZPT // CH) * CH, rem)])

      @pl.when(sid == NS - 1)
      def _dzlast():
        lb = (NS - 1) * ZPT
        for rep in range(LAST // CH):
          pltpu.sync_copy(ones_v, deg_sh.at[pl.ds(lb + rep * CH, CH)])
        rem = LAST - (LAST // CH) * CH
        if rem:
          pltpu.sync_copy(ones_v.at[pl.ds(0, rem)],
                          deg_sh.at[pl.ds(lb + (LAST // CH) * CH, rem)])

      pltpu.sync_copy(ones_h, ones_v)

    def stage_tab(tab_h):
      @pl.when(sid < NS - 1)
      def _sfull():
        pltpu.sync_copy(tab_h.at[cid].at[pl.ds(base, ZPT)],
                        tab_sh.at[pl.ds(base, ZPT)])

      @pl.when(sid == NS - 1)
      def _slast():
        pltpu.sync_copy(tab_h.at[cid].at[pl.ds((NS - 1) * ZPT, LAST)],
                        tab_sh.at[pl.ds((NS - 1) * ZPT, LAST)])

    stage_tab(a_tab_h)
    plsc.subcore_barrier()

    def burst_phase(gather_by_src, first):
      gidx_v = srcv if gather_by_src else dstv
      sidx_v = dstv if gather_by_src else srcv
      B = K * CH

      def issue_gathers(g, boff):
        for k in range(K):
          pltpu.async_copy(
              tab_sh.at[gidx_v.at[g * K + k]],
              rows_v.at[pl.ds(boff + k * CH, CH)], gsem)

      def wait_gathers(g, boff):
        for k in range(K):
          pltpu.make_async_copy(
              tab_sh.at[gidx_v.at[g * K + k]],
              rows_v.at[pl.ds(boff + k * CH, CH)], gsem).wait()

      def issue_scatters(g, boff):
        for k in range(K):
          j = g * K + k
          pltpu.async_copy(
              rows_v.at[pl.ds(boff + k * CH, CH)], acc_sh.at[sidx_v.at[j]],
              ssem, add=True)
          if with_deg and first:
            @pl.when(cid == 0)
            def _deg_dst():
              pltpu.async_copy(ones_v, deg_sh.at[dstv.at[j]], dsem, add=True)

            @pl.when(cid == 1)
            def _deg_src():
              pltpu.async_copy(ones_v, deg_sh.at[srcv.at[j]], dsem, add=True)

      def wait_scatters(g, boff):
        for k in range(K):
          pltpu.make_async_copy(
              rows_v.at[pl.ds(boff + k * CH, CH)],
              acc_sh.at[sidx_v.at[g * K + k]], ssem).wait()

      def stage_body(stg, carry):
        pltpu.sync_copy(src_h.at[sid, stg], srcv)
        pltpu.sync_copy(dst_h.at[sid, stg], dstv)

        issue_gathers(0, 0)
        wait_gathers(0, 0)
        issue_gathers(1, B)
        issue_scatters(0, 0)

        def body(g, carry2):
          boff = (g % 2) * B
          alt = B - boff
          wait_gathers(g, boff)
          wait_scatters(g - 1, alt)
          issue_gathers(g + 1, alt)
          issue_scatters(g, boff)
          return carry2

        lax.fori_loop(1, NBH_S - 1, body, 0)
        gl = NBH_S - 1
        boff_l = (gl % 2) * B
        wait_gathers(gl, boff_l)
        wait_scatters(gl - 1, B - boff_l)
        issue_scatters(gl, boff_l)
        wait_scatters(gl, boff_l)

        if with_deg and first:
          @pl.when(cid == 0)
          def _drain_dst():
            def w(j, carry2):
              pltpu.make_async_copy(ones_v, deg_sh.at[dstv.at[j]],
                                    dsem).wait()
              return carry2
            lax.fori_loop(0, NCH_S, w, 0)

          @pl.when(cid == 1)
          def _drain_src():
            def w(j, carry2):
              pltpu.make_async_copy(ones_v, deg_sh.at[srcv.at[j]],
                                    dsem).wait()
              return carry2
            lax.fori_loop(0, NCH_S, w, 0)
        return carry

      lax.fori_loop(0, NSTG, stage_body, 0)

    def write_out(out_h, from_sh, width_last_ok=True):
      @pl.when(sid < NS - 1)
      def _full():
        pltpu.sync_copy(from_sh.at[pl.ds(base, ZPT)],
                        out_h.at[cid, pl.ds(base, ZPT)])

      @pl.when(sid == NS - 1)
      def _last():
        pltpu.sync_copy(from_sh.at[pl.ds((NS - 1) * ZPT, LAST)],
                        out_h.at[cid, pl.ds((NS - 1) * ZPT, LAST)])

    burst_phase(gather_by_src=True, first=True)
    plsc.subcore_barrier()
    write_out(sp_h, acc_sh)
    if with_deg:
      write_out(deg_h, deg_sh)
    zero_acc()
    stage_tab(p_tab_h)
    plsc.subcore_barrier()

    burst_phase(gather_by_src=False, first=False)
    plsc.subcore_barrier()
    write_out(sa_h, acc_sh)

  return rnd


_sc_round1 = _make_sc_round(True)
_sc_round2 = _make_sc_round(False)


_BLK = 1000


def _combine1_body(sums_ref, deg_ref, prev_ref, sw_ref, new_ref, inv_ref):
  s = jnp.concatenate([sums_ref[0], sums_ref[1]], axis=1)
  deg = deg_ref[0, :, 0:1]
  inv = 1.0 / jnp.maximum(deg, 1.0)
  sw = sw_ref[0, 0]
  new = s * inv + jnp.concatenate([prev_ref[0], prev_ref[1]], axis=1) * sw
  new_ref[0] = new[:, :DH]
  new_ref[1] = new[:, DH:]
  inv_ref[...] = inv


def _combine1(sums, deg, prev_split, sw):
  grid = N_NODES // _BLK
  return pl.pallas_call(
      _combine1_body,
      grid=(grid,),
      in_specs=[
          pl.BlockSpec((NC, _BLK, DH), lambda i: (0, i, 0)),
          pl.BlockSpec((1, _BLK, 8), lambda i: (0, i, 0)),
          pl.BlockSpec((NC, _BLK, DH), lambda i: (0, i, 0)),
          pl.BlockSpec(memory_space=pltpu.SMEM),
      ],
      out_specs=[
          pl.BlockSpec((NC, _BLK, DH), lambda i: (0, i, 0)),
          pl.BlockSpec((_BLK, 1), lambda i: (i, 0)),
      ],
      out_shape=[
          jax.ShapeDtypeStruct((NC, N_NODES, DH), jnp.float32),
          jax.ShapeDtypeStruct((N_NODES, 1), jnp.float32),
      ],
  )(sums, deg, prev_split, sw)


def _combine2_body(sums_ref, prev_ref, inv_ref, sw_ref, out_ref):
  s = jnp.concatenate([sums_ref[0], sums_ref[1]], axis=1)
  sw = sw_ref[0, 0]
  prev = jnp.concatenate([prev_ref[0], prev_ref[1]], axis=1)
  out_ref[...] = s * inv_ref[...] + prev * sw


def _combine2(sums, prev_split, inv, sw):
  grid = N_NODES // _BLK
  return pl.pallas_call(
      _combine2_body,
      grid=(grid,),
      in_specs=[
          pl.BlockSpec((NC, _BLK, DH), lambda i: (0, i, 0)),
          pl.BlockSpec((NC, _BLK, DH), lambda i: (0, i, 0)),
          pl.BlockSpec((_BLK, 1), lambda i: (i, 0)),
          pl.BlockSpec(memory_space=pltpu.SMEM),
      ],
      out_specs=pl.BlockSpec((_BLK, D), lambda i: (i, 0)),
      out_shape=jax.ShapeDtypeStruct((N_NODES, D), jnp.float32),
  )(sums, prev_split, inv, sw)


def kernel(author_emb, paper_emb, edge_index, u_sw, i_sw):
  src = edge_index[0].reshape(NS, NSTG, NCH_S, CH)
  dst = edge_index[1].reshape(NS, NSTG, NCH_S, CH)
  a_tab = author_emb.reshape(N_NODES, NC, DH).transpose(1, 0, 2)
  p_tab = paper_emb.reshape(N_NODES, NC, DH).transpose(1, 0, 2)
  u = u_sw.reshape(1, 1)
  i = i_sw.reshape(1, 1)

  ones8 = jnp.ones((CH, 8), jnp.float32)
  zeros8 = jnp.zeros((CH, 8), jnp.float32)
  sums_p, sums_a, degs = _sc_round1(a_tab, p_tab, src, dst, ones8, zeros8)
  p1, inv_p = _combine1(sums_p, degs[0:1], p_tab, i)
  a1, inv_a = _combine1(sums_a, degs[1:2], a_tab, u)

  sums_p2, sums_a2 = _sc_round2(a1, p1, src, dst)
  p2 = _combine2(sums_p2, p1, inv_p, i)
  a2 = _combine2(sums_a2, a1, inv_a, u)
  return (a2, p2)


# speedup vs baseline: 1.2423x; 1.2423x over previous
import functools

import jax
import jax.numpy as jnp
from jax import lax
from jax.experimental import pallas as pl
from jax.experimental.pallas import tpu as pltpu
from jax.experimental.pallas import tpu_sc as plsc

N_NODES = 10000
N_EDGES = 320000
D = 128

NC = 2
NS = 16
DH = D // NC
EPT = N_EDGES // NS
CH = 80
K = 5
NCH = EPT // CH
NSTG = 10
NCH_S = NCH // NSTG
NBH_S = NCH_S // K
ACC_ROWS = N_NODES
ZPT = 632
LAST = N_NODES - (NS - 1) * ZPT
LANES = 16


def _make_sc_round(with_deg):
  mesh = plsc.VectorSubcoreMesh(core_axis_name="c", subcore_axis_name="s")

  sum_t = jax.ShapeDtypeStruct((NC, N_NODES, DH), jnp.float32)
  out_type = (sum_t, sum_t)
  scratch = [
      pltpu.VMEM((NCH_S, CH), jnp.int32),
      pltpu.VMEM((NCH_S, CH), jnp.int32),
      pltpu.VMEM((2 * K * CH, DH), jnp.float32),
      pltpu.VMEM_SHARED((ACC_ROWS, DH), jnp.float32),
      pltpu.SemaphoreType.DMA,
      pltpu.SemaphoreType.DMA,
  ]
  if with_deg:
    out_type = out_type + (
        jax.ShapeDtypeStruct((NC, N_NODES, 8), jnp.float32),)
    scratch = scratch + [
        pltpu.VMEM((CH, 8), jnp.float32),
        pltpu.VMEM_SHARED((ACC_ROWS, 8), jnp.float32),
        pltpu.SemaphoreType.DMA,
    ]

  @functools.partial(
      pl.kernel, out_type=out_type, mesh=mesh, scratch_types=scratch,
      compiler_params=pltpu.CompilerParams(use_tc_tiling_on_sc=False))
  def rnd(a_tab_h, p_tab_h, src_h, dst_h, *refs):
    if with_deg:
      (ones_h, zeros_h, sp_h, sa_h, deg_h, srcv, dstv, rows_v, acc_sh,
       gsem, ssem, ones_v, deg_sh, dsem) = refs
    else:
      sp_h, sa_h, srcv, dstv, rows_v, acc_sh, gsem, ssem = refs
    cid = lax.axis_index("c")
    sid = lax.axis_index("s")

    zvec = jnp.zeros((LANES,), jnp.float32)
    base = sid * ZPT

    def zero_acc():
      def zrow(r, carry):
        for j in range(DH // LANES):
          rows_v[r, pl.ds(j * LANES, LANES)] = zvec
        return carry

      lax.fori_loop(0, ZPT, zrow, 0)

      @pl.when(sid < NS - 1)
      def _zfull():
        pltpu.sync_copy(rows_v.at[pl.ds(0, ZPT)],
                        acc_sh.at[pl.ds(base, ZPT)])

      @pl.when(sid == NS - 1)
      def _zlast():
        pltpu.sync_copy(rows_v.at[pl.ds(0, LAST)],
                        acc_sh.at[pl.ds((NS - 1) * ZPT, LAST)])

    zero_acc()
    if with_deg:
      pltpu.sync_copy(zeros_h, ones_v)

      @pl.when(sid < NS - 1)
      def _dzfull():
        for rep in range(ZPT // CH):
          pltpu.sync_copy(ones_v, deg_sh.at[pl.ds(base + rep * CH, CH)])
        rem = ZPT - (ZPT // CH) * CH
        if rem:
          pltpu.sync_copy(ones_v.at[pl.ds(0, rem)],
                          deg_sh.at[pl.ds(base + (ZPT // CH) * CH, rem)])

      @pl.when(sid == NS - 1)
      def _dzlast():
        lb = (NS - 1) * ZPT
        for rep in range(LAST // CH):
          pltpu.sync_copy(ones_v, deg_sh.at[pl.ds(lb + rep * CH, CH)])
        rem = LAST - (LAST // CH) * CH
        if rem:
          pltpu.sync_copy(ones_v.at[pl.ds(0, rem)],
                          deg_sh.at[pl.ds(lb + (LAST // CH) * CH, rem)])

      pltpu.sync_copy(ones_h, ones_v)
    plsc.subcore_barrier()

    def burst_phase(gather_by_src, first):
      gtab_h = a_tab_h if gather_by_src else p_tab_h
      gidx_v = srcv if gather_by_src else dstv
      sidx_v = dstv if gather_by_src else srcv
      B = K * CH

      def issue_gathers(g, boff):
        for k in range(K):
          pltpu.async_copy(
              gtab_h.at[cid].at[gidx_v.at[g * K + k]],
              rows_v.at[pl.ds(boff + k * CH, CH)], gsem)

      def wait_gathers(g, boff):
        for k in range(K):
          pltpu.make_async_copy(
              gtab_h.at[cid].at[gidx_v.at[g * K + k]],
              rows_v.at[pl.ds(boff + k * CH, CH)], gsem).wait()

      def issue_scatters(g, boff):
        for k in range(K):
          j = g * K + k
          pltpu.async_copy(
              rows_v.at[pl.ds(boff + k * CH, CH)], acc_sh.at[sidx_v.at[j]],
              ssem, add=True)
          if with_deg and first:
            @pl.when(cid == 0)
            def _deg_dst():
              pltpu.async_copy(ones_v, deg_sh.at[dstv.at[j]], dsem, add=True)

            @pl.when(cid == 1)
            def _deg_src():
              pltpu.async_copy(ones_v, deg_sh.at[srcv.at[j]], dsem, add=True)

      def wait_scatters(g, boff):
        for k in range(K):
          pltpu.make_async_copy(
              rows_v.at[pl.ds(boff + k * CH, CH)],
              acc_sh.at[sidx_v.at[g * K + k]], ssem).wait()

      def stage_body(stg, carry):
        pltpu.sync_copy(src_h.at[sid, stg], srcv)
        pltpu.sync_copy(dst_h.at[sid, stg], dstv)

        issue_gathers(0, 0)
        wait_gathers(0, 0)
        issue_gathers(1, B)
        issue_scatters(0, 0)

        def body(g, carry2):
          boff = (g % 2) * B
          alt = B - boff
          wait_gathers(g, boff)
          wait_scatters(g - 1, alt)
          issue_gathers(g + 1, alt)
          issue_scatters(g, boff)
          return carry2

        lax.fori_loop(1, NBH_S - 1, body, 0)
        gl = NBH_S - 1
        boff_l = (gl % 2) * B
        wait_gathers(gl, boff_l)
        wait_scatters(gl - 1, B - boff_l)
        issue_scatters(gl, boff_l)
        wait_scatters(gl, boff_l)

        if with_deg and first:
          @pl.when(cid == 0)
          def _drain_dst():
            def w(j, carry2):
              pltpu.make_async_copy(ones_v, deg_sh.at[dstv.at[j]],
                                    dsem).wait()
              return carry2
            lax.fori_loop(0, NCH_S, w, 0)

          @pl.when(cid == 1)
          def _drain_src():
            def w(j, carry2):
              pltpu.make_async_copy(ones_v, deg_sh.at[srcv.at[j]],
                                    dsem).wait()
              return carry2
            lax.fori_loop(0, NCH_S, w, 0)
        return carry

      lax.fori_loop(0, NSTG, stage_body, 0)

    def write_out(out_h, from_sh, width_last_ok=True):
      @pl.when(sid < NS - 1)
      def _full():
        pltpu.sync_copy(from_sh.at[pl.ds(base, ZPT)],
                        out_h.at[cid, pl.ds(base, ZPT)])

      @pl.when(sid == NS - 1)
      def _last():
        pltpu.sync_copy(from_sh.at[pl.ds((NS - 1) * ZPT, LAST)],
                        out_h.at[cid, pl.ds((NS - 1) * ZPT, LAST)])

    burst_phase(gather_by_src=True, first=True)
    plsc.subcore_barrier()
    write_out(sp_h, acc_sh)
    if with_deg:
      write_out(deg_h, deg_sh)
    zero_acc()
    plsc.subcore_barrier()

    burst_phase(gather_by_src=False, first=False)
    plsc.subcore_barrier()
    write_out(sa_h, acc_sh)

  return rnd


_sc_round1 = _make_sc_round(True)
_sc_round2 = _make_sc_round(False)


_BLK = 1000


def _combine1_body(sums_ref, deg_ref, prev_ref, sw_ref, new_ref, inv_ref):
  s = jnp.concatenate([sums_ref[0], sums_ref[1]], axis=1)
  deg = deg_ref[0, :, 0:1]
  inv = 1.0 / jnp.maximum(deg, 1.0)
  sw = sw_ref[0, 0]
  new = s * inv + jnp.concatenate([prev_ref[0], prev_ref[1]], axis=1) * sw
  new_ref[0] = new[:, :DH]
  new_ref[1] = new[:, DH:]
  inv_ref[...] = inv


def _combine1(sums, deg, prev_split, sw):
  grid = N_NODES // _BLK
  return pl.pallas_call(
      _combine1_body,
      grid=(grid,),
      in_specs=[
          pl.BlockSpec((NC, _BLK, DH), lambda i: (0, i, 0)),
          pl.BlockSpec((1, _BLK, 8), lambda i: (0, i, 0)),
          pl.BlockSpec((NC, _BLK, DH), lambda i: (0, i, 0)),
          pl.BlockSpec(memory_space=pltpu.SMEM),
      ],
      out_specs=[
          pl.BlockSpec((NC, _BLK, DH), lambda i: (0, i, 0)),
          pl.BlockSpec((_BLK, 1), lambda i: (i, 0)),
      ],
      out_shape=[
          jax.ShapeDtypeStruct((NC, N_NODES, DH), jnp.float32),
          jax.ShapeDtypeStruct((N_NODES, 1), jnp.float32),
      ],
  )(sums, deg, prev_split, sw)


def _combine2_body(sums_ref, prev_ref, inv_ref, sw_ref, out_ref):
  s = jnp.concatenate([sums_ref[0], sums_ref[1]], axis=1)
  sw = sw_ref[0, 0]
  prev = jnp.concatenate([prev_ref[0], prev_ref[1]], axis=1)
  out_ref[...] = s * inv_ref[...] + prev * sw


def _combine2(sums, prev_split, inv, sw):
  grid = N_NODES // _BLK
  return pl.pallas_call(
      _combine2_body,
      grid=(grid,),
      in_specs=[
          pl.BlockSpec((NC, _BLK, DH), lambda i: (0, i, 0)),
          pl.BlockSpec((NC, _BLK, DH), lambda i: (0, i, 0)),
          pl.BlockSpec((_BLK, 1), lambda i: (i, 0)),
          pl.BlockSpec(memory_space=pltpu.SMEM),
      ],
      out_specs=pl.BlockSpec((_BLK, D), lambda i: (i, 0)),
      out_shape=jax.ShapeDtypeStruct((N_NODES, D), jnp.float32),
  )(sums, prev_split, inv, sw)


def kernel(author_emb, paper_emb, edge_index, u_sw, i_sw):
  src = edge_index[0].reshape(NS, NSTG, NCH_S, CH)
  dst = edge_index[1].reshape(NS, NSTG, NCH_S, CH)
  a_tab = author_emb.reshape(N_NODES, NC, DH).transpose(1, 0, 2)
  p_tab = paper_emb.reshape(N_NODES, NC, DH).transpose(1, 0, 2)
  u = u_sw.reshape(1, 1)
  i = i_sw.reshape(1, 1)

  ones8 = jnp.ones((CH, 8), jnp.float32)
  zeros8 = jnp.zeros((CH, 8), jnp.float32)
  sums_p, sums_a, degs = _sc_round1(a_tab, p_tab, src, dst, ones8, zeros8)
  p1, inv_p = _combine1(sums_p, degs[0:1], p_tab, i)
  a1, inv_a = _combine1(sums_a, degs[1:2], a_tab, u)

  sums_p2, sums_a2 = _sc_round2(a1, p1, src, dst)
  p2 = _combine2(sums_p2, p1, inv_p, i)
  a2 = _combine2(sums_a2, a1, inv_a, u)
  return (a2, p2)


# speedup vs baseline: 1.3501x; 1.0867x over previous
import functools

import jax
import jax.numpy as jnp
from jax import lax
from jax.experimental import pallas as pl
from jax.experimental.pallas import tpu as pltpu
from jax.experimental.pallas import tpu_sc as plsc

N_NODES = 10000
N_EDGES = 320000
D = 128

NC = 2
NS = 16
DH = D // NC
EPT = N_EDGES // NS
CH = 80
K = 5
NCH = EPT // CH
NCH_H = NCH // 2
NBH = NCH_H // K
ACC_ROWS = N_NODES
ZPT = 632
LAST = N_NODES - (NS - 1) * ZPT
LANES = 16


def _make_sc_round(with_deg):
  mesh = plsc.VectorSubcoreMesh(core_axis_name="c", subcore_axis_name="s")

  sum_t = jax.ShapeDtypeStruct((NC, N_NODES, DH), jnp.float32)
  out_type = (sum_t, sum_t)
  scratch = [
      pltpu.VMEM((NCH_H, CH), jnp.int32),
      pltpu.VMEM((NCH_H, CH), jnp.int32),
      pltpu.VMEM((2 * K * CH, DH), jnp.float32),
      pltpu.VMEM_SHARED((ACC_ROWS, DH), jnp.float32),
      pltpu.SemaphoreType.DMA,
      pltpu.SemaphoreType.DMA,
  ]
  if with_deg:
    out_type = out_type + (
        jax.ShapeDtypeStruct((NC, N_NODES, 8), jnp.float32),)
    scratch = scratch + [
        pltpu.VMEM((CH, 8), jnp.float32),
        pltpu.VMEM_SHARED((ACC_ROWS, 8), jnp.float32),
        pltpu.SemaphoreType.DMA,
    ]

  @functools.partial(
      pl.kernel, out_type=out_type, mesh=mesh, scratch_types=scratch,
      compiler_params=pltpu.CompilerParams(use_tc_tiling_on_sc=False))
  def rnd(a_tab_h, p_tab_h, src_h, dst_h, *refs):
    if with_deg:
      (ones_h, zeros_h, sp_h, sa_h, deg_h, srcv, dstv, rows_v, acc_sh,
       gsem, ssem, ones_v, deg_sh, dsem) = refs
    else:
      sp_h, sa_h, srcv, dstv, rows_v, acc_sh, gsem, ssem = refs
    cid = lax.axis_index("c")
    sid = lax.axis_index("s")

    zvec = jnp.zeros((LANES,), jnp.float32)
    base = sid * ZPT

    def zero_acc():
      def zrow(r, carry):
        for j in range(DH // LANES):
          rows_v[r, pl.ds(j * LANES, LANES)] = zvec
        return carry

      lax.fori_loop(0, ZPT, zrow, 0)

      @pl.when(sid < NS - 1)
      def _zfull():
        pltpu.sync_copy(rows_v.at[pl.ds(0, ZPT)],
                        acc_sh.at[pl.ds(base, ZPT)])

      @pl.when(sid == NS - 1)
      def _zlast():
        pltpu.sync_copy(rows_v.at[pl.ds(0, LAST)],
                        acc_sh.at[pl.ds((NS - 1) * ZPT, LAST)])

    zero_acc()
    if with_deg:
      pltpu.sync_copy(zeros_h, ones_v)

      @pl.when(sid < NS - 1)
      def _dzfull():
        for rep in range(ZPT // CH):
          pltpu.sync_copy(ones_v, deg_sh.at[pl.ds(base + rep * CH, CH)])
        rem = ZPT - (ZPT // CH) * CH
        pltpu.sync_copy(ones_v.at[pl.ds(0, rem)],
                        deg_sh.at[pl.ds(base + (ZPT // CH) * CH, rem)])

      @pl.when(sid == NS - 1)
      def _dzlast():
        lb = (NS - 1) * ZPT
        for rep in range(LAST // CH):
          pltpu.sync_copy(ones_v, deg_sh.at[pl.ds(lb + rep * CH, CH)])
        rem = LAST - (LAST // CH) * CH
        pltpu.sync_copy(ones_v.at[pl.ds(0, rem)],
                        deg_sh.at[pl.ds(lb + (LAST // CH) * CH, rem)])

      pltpu.sync_copy(ones_h, ones_v)
    plsc.subcore_barrier()

    def burst_phase(gather_by_src, first):
      gtab_h = a_tab_h if gather_by_src else p_tab_h
      gidx_v = srcv if gather_by_src else dstv
      sidx_v = dstv if gather_by_src else srcv
      B = K * CH

      def issue_gathers(g, boff):
        for k in range(K):
          pltpu.async_copy(
              gtab_h.at[cid].at[gidx_v.at[g * K + k]],
              rows_v.at[pl.ds(boff + k * CH, CH)], gsem)

      def wait_gathers(g, boff):
        for k in range(K):
          pltpu.make_async_copy(
              gtab_h.at[cid].at[gidx_v.at[g * K + k]],
              rows_v.at[pl.ds(boff + k * CH, CH)], gsem).wait()

      def issue_scatters(g, boff):
        for k in range(K):
          j = g * K + k
          pltpu.async_copy(
              rows_v.at[pl.ds(boff + k * CH, CH)], acc_sh.at[sidx_v.at[j]],
              ssem, add=True)
          if with_deg and first:
            @pl.when(cid == 0)
            def _deg_dst():
              pltpu.async_copy(ones_v, deg_sh.at[dstv.at[j]], dsem, add=True)

            @pl.when(cid == 1)
            def _deg_src():
              pltpu.async_copy(ones_v, deg_sh.at[srcv.at[j]], dsem, add=True)

      def wait_scatters(g, boff):
        for k in range(K):
          pltpu.make_async_copy(
              rows_v.at[pl.ds(boff + k * CH, CH)],
              acc_sh.at[sidx_v.at[g * K + k]], ssem).wait()

      for half in range(2):
        pltpu.sync_copy(src_h.at[sid, half], srcv)
        pltpu.sync_copy(dst_h.at[sid, half], dstv)

        issue_gathers(0, 0)
        wait_gathers(0, 0)
        issue_gathers(1, B)
        issue_scatters(0, 0)

        def body(g, carry):
          boff = (g % 2) * B
          alt = B - boff
          wait_gathers(g, boff)
          wait_scatters(g - 1, alt)
          issue_gathers(g + 1, alt)
          issue_scatters(g, boff)
          return carry

        lax.fori_loop(1, NBH - 1, body, 0)
        gl = NBH - 1
        boff_l = (gl % 2) * B
        wait_gathers(gl, boff_l)
        wait_scatters(gl - 1, B - boff_l)
        issue_scatters(gl, boff_l)
        wait_scatters(gl, boff_l)

        if with_deg and first:
          @pl.when(cid == 0)
          def _drain_dst():
            def w(j, carry):
              pltpu.make_async_copy(ones_v, deg_sh.at[dstv.at[j]],
                                    dsem).wait()
              return carry
            lax.fori_loop(0, NCH_H, w, 0)

          @pl.when(cid == 1)
          def _drain_src():
            def w(j, carry):
              pltpu.make_async_copy(ones_v, deg_sh.at[srcv.at[j]],
                                    dsem).wait()
              return carry
            lax.fori_loop(0, NCH_H, w, 0)

    def write_out(out_h, from_sh, width_last_ok=True):
      @pl.when(sid < NS - 1)
      def _full():
        pltpu.sync_copy(from_sh.at[pl.ds(base, ZPT)],
                        out_h.at[cid, pl.ds(base, ZPT)])

      @pl.when(sid == NS - 1)
      def _last():
        pltpu.sync_copy(from_sh.at[pl.ds((NS - 1) * ZPT, LAST)],
                        out_h.at[cid, pl.ds((NS - 1) * ZPT, LAST)])

    burst_phase(gather_by_src=True, first=True)
    plsc.subcore_barrier()
    write_out(sp_h, acc_sh)
    if with_deg:
      write_out(deg_h, deg_sh)
    zero_acc()
    plsc.subcore_barrier()

    burst_phase(gather_by_src=False, first=False)
    plsc.subcore_barrier()
    write_out(sa_h, acc_sh)

  return rnd


_sc_round1 = _make_sc_round(True)
_sc_round2 = _make_sc_round(False)


_BLK = 1000


def _combine1_body(sums_ref, deg_ref, prev_ref, sw_ref, new_ref, inv_ref):
  s = jnp.concatenate([sums_ref[0], sums_ref[1]], axis=1)
  deg = deg_ref[0, :, 0:1]
  inv = 1.0 / jnp.maximum(deg, 1.0)
  sw = sw_ref[0, 0]
  new = s * inv + jnp.concatenate([prev_ref[0], prev_ref[1]], axis=1) * sw
  new_ref[0] = new[:, :DH]
  new_ref[1] = new[:, DH:]
  inv_ref[...] = inv


def _combine1(sums, deg, prev_split, sw):
  grid = N_NODES // _BLK
  return pl.pallas_call(
      _combine1_body,
      grid=(grid,),
      in_specs=[
          pl.BlockSpec((NC, _BLK, DH), lambda i: (0, i, 0)),
          pl.BlockSpec((1, _BLK, 8), lambda i: (0, i, 0)),
          pl.BlockSpec((NC, _BLK, DH), lambda i: (0, i, 0)),
          pl.BlockSpec(memory_space=pltpu.SMEM),
      ],
      out_specs=[
          pl.BlockSpec((NC, _BLK, DH), lambda i: (0, i, 0)),
          pl.BlockSpec((_BLK, 1), lambda i: (i, 0)),
      ],
      out_shape=[
          jax.ShapeDtypeStruct((NC, N_NODES, DH), jnp.float32),
          jax.ShapeDtypeStruct((N_NODES, 1), jnp.float32),
      ],
  )(sums, deg, prev_split, sw)


def _combine2_body(sums_ref, prev_ref, inv_ref, sw_ref, out_ref):
  s = jnp.concatenate([sums_ref[0], sums_ref[1]], axis=1)
  sw = sw_ref[0, 0]
  prev = jnp.concatenate([prev_ref[0], prev_ref[1]], axis=1)
  out_ref[...] = s * inv_ref[...] + prev * sw


def _combine2(sums, prev_split, inv, sw):
  grid = N_NODES // _BLK
  return pl.pallas_call(
      _combine2_body,
      grid=(grid,),
      in_specs=[
          pl.BlockSpec((NC, _BLK, DH), lambda i: (0, i, 0)),
          pl.BlockSpec((NC, _BLK, DH), lambda i: (0, i, 0)),
          pl.BlockSpec((_BLK, 1), lambda i: (i, 0)),
          pl.BlockSpec(memory_space=pltpu.SMEM),
      ],
      out_specs=pl.BlockSpec((_BLK, D), lambda i: (i, 0)),
      out_shape=jax.ShapeDtypeStruct((N_NODES, D), jnp.float32),
  )(sums, prev_split, inv, sw)


def kernel(author_emb, paper_emb, edge_index, u_sw, i_sw):
  src = edge_index[0].reshape(NS, 2, NCH_H, CH)
  dst = edge_index[1].reshape(NS, 2, NCH_H, CH)
  a_tab = author_emb.reshape(N_NODES, NC, DH).transpose(1, 0, 2)
  p_tab = paper_emb.reshape(N_NODES, NC, DH).transpose(1, 0, 2)
  u = u_sw.reshape(1, 1)
  i = i_sw.reshape(1, 1)

  ones8 = jnp.ones((CH, 8), jnp.float32)
  zeros8 = jnp.zeros((CH, 8), jnp.float32)
  sums_p, sums_a, degs = _sc_round1(a_tab, p_tab, src, dst, ones8, zeros8)
  p1, inv_p = _combine1(sums_p, degs[0:1], p_tab, i)
  a1, inv_a = _combine1(sums_a, degs[1:2], a_tab, u)

  sums_p2, sums_a2 = _sc_round2(a1, p1, src, dst)
  p2 = _combine2(sums_p2, p1, inv_p, i)
  a2 = _combine2(sums_a2, a1, inv_a, u)
  return (a2, p2)
